# Initial kernel scaffold; baseline (speedup 1.0000x reference)
import jax
import jax.numpy as jnp
from jax.experimental import pallas as pl


def kernel(input_ids, token_embed, pos_embed):
    raise NotImplementedError("write your pallas kernel here")


# speedup vs baseline: 2.2464x; 2.2464x over previous
import functools

import jax
import jax.numpy as jnp
from jax import lax
from jax.experimental import pallas as pl
from jax.experimental.pallas import tpu as pltpu
from jax.experimental.pallas import tpu_sc as plsc

_BATCH = 4096
_SEQ = 128
_EMBED = 64
_ROWS = _BATCH * _SEQ
_NC = 2
_NS = 16
_NW = _NC * _NS
_NPW = _ROWS // _NW
_C = 128
_NCH = _NPW // _C
_LG = _EMBED // 16


def _body(ids_hbm, tok_hbm, pos_hbm, out_hbm,
          idx_all, row0, row1, out0, out1, posb,
          gsem0, gsem1, osem0, osem1):
    wid = lax.axis_index("s") * _NC + lax.axis_index("c")
    base = wid * _NPW

    rows = (row0, row1)
    outs = (out0, out1)
    gsems = (gsem0, gsem1)
    osems = (osem0, osem1)

    pltpu.sync_copy(ids_hbm.at[pl.ds(base, _NPW)], idx_all)
    pltpu.sync_copy(pos_hbm, posb)

    def gather(c, b):
        return pltpu.async_copy(
            tok_hbm.at[idx_all.at[pl.ds(c * _C, _C)]], rows[b], gsems[b])

    def outcopy(c, b):
        return pltpu.make_async_copy(
            outs[b], out_hbm.at[pl.ds(base + c * _C, _C)], osems[b])

    gather(0, 0)
    gather(1, 1)

    def chunk_pair(i, carry):
        for b in range(2):
            c = i * 2 + b
            pltpu.make_async_copy(
                tok_hbm.at[idx_all.at[pl.ds(c * _C, _C)]],
                rows[b], gsems[b]).wait()

            @pl.when(i >= 1)
            def _():
                outcopy(c - 2, b).wait()

            rb = rows[b]
            ob = outs[b]

            @plsc.parallel_loop(0, _C, step=1, unroll=2)
            def _(r):
                for g in range(_LG):
                    sl = pl.ds(g * 16, 16)
                    ob[r, sl] = rb[r, sl] + posb[r, sl]

            outcopy(c, b).start()

            @pl.when(i < _NCH // 2 - 1)
            def _():
                gather(c + 2, b)
        return carry

    lax.fori_loop(0, _NCH // 2, chunk_pair, 0)

    outcopy(_NCH - 2, 0).wait()
    outcopy(_NCH - 1, 1).wait()


@jax.jit
def _embed_lookup(ids_flat, token_embed, pos_embed):
    mesh = plsc.VectorSubcoreMesh(core_axis_name="c", subcore_axis_name="s",
                                  num_cores=_NC, num_subcores=_NS)
    return pl.kernel(
        _body,
        out_type=jax.ShapeDtypeStruct((_ROWS, _EMBED), jnp.float32),
        mesh=mesh,
        scratch_types=[
            pltpu.VMEM((_NPW,), jnp.int32),
            pltpu.VMEM((_C, _EMBED), jnp.float32),
            pltpu.VMEM((_C, _EMBED), jnp.float32),
            pltpu.VMEM((_C, _EMBED), jnp.float32),
            pltpu.VMEM((_C, _EMBED), jnp.float32),
            pltpu.VMEM((_SEQ, _EMBED), jnp.float32),
            pltpu.SemaphoreType.DMA,
            pltpu.SemaphoreType.DMA,
            pltpu.SemaphoreType.DMA,
            pltpu.SemaphoreType.DMA,
        ],
        compiler_params=pltpu.CompilerParams(use_tc_tiling_on_sc=False),
    )(ids_flat, token_embed, pos_embed)


def kernel(input_ids, token_embed, pos_embed):
    ids_flat = input_ids.reshape(-1).astype(jnp.int32)
    out = _embed_lookup(ids_flat, token_embed, pos_embed)
    return out.reshape(_BATCH, _SEQ, _EMBED)
